# baseline (speedup 1.0000x reference)
import functools

import jax
import jax.numpy as jnp
from jax import lax
from jax.experimental import pallas as pl
from jax.experimental.pallas import tpu as pltpu
from jax.experimental.pallas import tpu_sc as plsc


def _ring_steps(nchunks, nb, get, put):
    state = {"gets": [], "writes": [None] * nchunks}

    def prime():
        state["gets"] = [get(j) for j in range(min(nb - 1, nchunks))]

    def step(j):
        if j >= nchunks:
            return
        if j + nb - 1 < nchunks:
            if j >= 1:
                state["writes"][j - 1].wait()
            state["gets"].append(get(j + nb - 1))
        state["gets"][j].wait()
        state["writes"][j] = put(j)

    def drain():
        for j in range(max(0, nchunks - nb), nchunks):
            state["writes"][j].wait()

    return prime, step, drain


def _copy_rows(n, d, nw, rows_per_w, ch, split):
    mesh = plsc.VectorSubcoreMesh(core_axis_name="c", subcore_axis_name="s")
    nb = 3
    na_chunks = split // ch
    nbc_chunks = (rows_per_w - split) // ch

    @functools.partial(
        pl.kernel,
        mesh=mesh,
        out_type=jax.ShapeDtypeStruct((n, d), jnp.float32),
        scratch_types=[
            pltpu.VMEM_SHARED((16, nb, ch, d), jnp.float32),
            pltpu.VMEM((ch, d), jnp.float32),
            pltpu.VMEM((ch, d), jnp.float32),
            pltpu.VMEM((ch, d), jnp.float32),
            pltpu.SemaphoreType.DMA,
            pltpu.SemaphoreType.DMA,
            pltpu.SemaphoreType.DMA,
            pltpu.SemaphoreType.DMA,
        ],
    )
    def k(z_hbm, out_hbm, spb, t0, t1, t2, sga, swa, sgb, swb):
        wid = lax.axis_index("s") * 2 + lax.axis_index("c")
        sid = lax.axis_index("s")
        base_a = wid * rows_per_w
        base_b = base_a + split
        tb = (t0, t1, t2)

        def a_get(j):
            return pltpu.async_copy(
                z_hbm.at[pl.ds(base_a + j * ch, ch)],
                spb.at[sid].at[j % nb], sga)

        def a_put(j):
            return pltpu.async_copy(
                spb.at[sid].at[j % nb],
                out_hbm.at[pl.ds(base_a + j * ch, ch)], swa)

        def b_get(j):
            return pltpu.async_copy(
                z_hbm.at[pl.ds(base_b + j * ch, ch)], tb[j % nb], sgb)

        def b_put(j):
            return pltpu.async_copy(
                tb[j % nb], out_hbm.at[pl.ds(base_b + j * ch, ch)], swb)

        a_prime, a_step, a_drain = _ring_steps(na_chunks, nb, a_get, a_put)
        b_prime, b_step, b_drain = _ring_steps(nbc_chunks, nb, b_get, b_put)
        a_prime()
        b_prime()
        for j in range(max(na_chunks, nbc_chunks)):
            a_step(j)
            b_step(j)
        a_drain()
        b_drain()

    return k


def _permute_rows(n, d, nw, nchunks, ch):
    mesh = plsc.VectorSubcoreMesh(core_axis_name="c", subcore_axis_name="s")
    rows_per_w = nchunks * ch

    @functools.partial(
        pl.kernel,
        mesh=mesh,
        out_type=jax.ShapeDtypeStruct((n, d), jnp.float32),
        scratch_types=[
            pltpu.VMEM((nchunks, ch), jnp.int32),
            pltpu.VMEM((ch, d), jnp.float32),
            pltpu.VMEM((ch, d), jnp.float32),
            pltpu.VMEM((ch, d), jnp.float32),
            pltpu.SemaphoreType.DMA,
            pltpu.SemaphoreType.DMA,
        ],
    )
    def k(z_hbm, idx3_hbm, out_hbm, idx_v, buf0, buf1, buf2, sem_g, sem_w):
        wid = lax.axis_index("s") * 2 + lax.axis_index("c")
        row_base = wid * rows_per_w

        pltpu.sync_copy(idx3_hbm.at[wid], idx_v)
        bufs = (buf0, buf1, buf2)
        nb = len(bufs)

        def gather(j):
            return pltpu.async_copy(z_hbm.at[idx_v.at[j]], bufs[j % nb], sem_g)

        def put(j):
            return pltpu.async_copy(
                bufs[j % nb],
                out_hbm.at[pl.ds(row_base + j * ch, ch)],
                sem_w,
            )

        gathers = [gather(j) for j in range(min(nb - 1, nchunks))]
        writes = [None] * nchunks
        for j in range(nchunks):
            if j + nb - 1 < nchunks:
                if j >= 1:
                    writes[j - 1].wait()
                gathers.append(gather(j + nb - 1))
            gathers[j].wait()
            writes[j] = put(j)
        for j in range(max(0, nchunks - nb), nchunks):
            writes[j].wait()

    return k


def kernel(z, indices):
    b, c, d = z.shape
    n = b * c
    info = plsc.get_sparse_core_info()
    nw = info.num_cores * info.num_subcores
    ch = 4
    nchunks = n // (nw * ch)
    rows_per_w = nchunks * ch
    row_idx = (jnp.arange(b, dtype=jnp.int32) * c)[:, None] + indices[None, :]
    idx3 = row_idx.reshape(nw, nchunks, ch)
    zf = z.reshape(n, d)
    is_id = jnp.all(indices == jnp.arange(c, dtype=indices.dtype))
    out = lax.cond(
        is_id,
        lambda: _copy_rows(n, d, nw, rows_per_w, 2, split=rows_per_w // 2)(zf),
        lambda: _permute_rows(n, d, nw, nchunks, ch)(zf, idx3),
    )
    return out.reshape(b, c, d)


# speedup vs baseline: 1.0642x; 1.0642x over previous
import functools

import jax
import jax.numpy as jnp
from jax import lax
from jax.experimental import pallas as pl
from jax.experimental.pallas import tpu as pltpu
from jax.experimental.pallas import tpu_sc as plsc


def _ring_steps(nchunks, nb, get, put):
    state = {"gets": [], "writes": [None] * nchunks}

    def prime():
        state["gets"] = [get(j) for j in range(min(nb - 1, nchunks))]

    def step(j):
        if j >= nchunks:
            return
        if j + nb - 1 < nchunks:
            if j >= 1:
                state["writes"][j - 1].wait()
            state["gets"].append(get(j + nb - 1))
        state["gets"][j].wait()
        state["writes"][j] = put(j)

    def drain():
        for j in range(max(0, nchunks - nb), nchunks):
            state["writes"][j].wait()

    return prime, step, drain


def _copy_rows(n, d, nw, rows_per_w, ch, nb):
    mesh = plsc.VectorSubcoreMesh(core_axis_name="c", subcore_axis_name="s")
    nchunks = rows_per_w // ch

    @functools.partial(
        pl.kernel,
        mesh=mesh,
        out_type=jax.ShapeDtypeStruct((n, d), jnp.float32),
        scratch_types=[
            pltpu.VMEM_SHARED((16, nb, ch, d), jnp.float32),
            pltpu.SemaphoreType.DMA,
            pltpu.SemaphoreType.DMA,
        ],
    )
    def k(z_hbm, out_hbm, spb, sem_g, sem_w):
        wid = lax.axis_index("s") * 2 + lax.axis_index("c")
        sid = lax.axis_index("s")
        row_base = wid * rows_per_w

        def get(j):
            return pltpu.async_copy(
                z_hbm.at[pl.ds(row_base + j * ch, ch)],
                spb.at[sid].at[j % nb], sem_g)

        def put(j):
            return pltpu.async_copy(
                spb.at[sid].at[j % nb],
                out_hbm.at[pl.ds(row_base + j * ch, ch)], sem_w)

        prime, step, drain = _ring_steps(nchunks, nb, get, put)
        prime()
        for j in range(nchunks):
            step(j)
        drain()

    return k


def _permute_rows(n, d, nw, nchunks, ch):
    mesh = plsc.VectorSubcoreMesh(core_axis_name="c", subcore_axis_name="s")
    rows_per_w = nchunks * ch

    @functools.partial(
        pl.kernel,
        mesh=mesh,
        out_type=jax.ShapeDtypeStruct((n, d), jnp.float32),
        scratch_types=[
            pltpu.VMEM((nchunks, ch), jnp.int32),
            pltpu.VMEM((ch, d), jnp.float32),
            pltpu.VMEM((ch, d), jnp.float32),
            pltpu.VMEM((ch, d), jnp.float32),
            pltpu.SemaphoreType.DMA,
            pltpu.SemaphoreType.DMA,
        ],
    )
    def k(z_hbm, idx3_hbm, out_hbm, idx_v, buf0, buf1, buf2, sem_g, sem_w):
        wid = lax.axis_index("s") * 2 + lax.axis_index("c")
        row_base = wid * rows_per_w

        pltpu.sync_copy(idx3_hbm.at[wid], idx_v)
        bufs = (buf0, buf1, buf2)
        nb = len(bufs)

        def gather(j):
            return pltpu.async_copy(z_hbm.at[idx_v.at[j]], bufs[j % nb], sem_g)

        def put(j):
            return pltpu.async_copy(
                bufs[j % nb],
                out_hbm.at[pl.ds(row_base + j * ch, ch)],
                sem_w,
            )

        gathers = [gather(j) for j in range(min(nb - 1, nchunks))]
        writes = [None] * nchunks
        for j in range(nchunks):
            if j + nb - 1 < nchunks:
                if j >= 1:
                    writes[j - 1].wait()
                gathers.append(gather(j + nb - 1))
            gathers[j].wait()
            writes[j] = put(j)
        for j in range(max(0, nchunks - nb), nchunks):
            writes[j].wait()

    return k


def kernel(z, indices):
    b, c, d = z.shape
    n = b * c
    info = plsc.get_sparse_core_info()
    nw = info.num_cores * info.num_subcores
    ch = 4
    nchunks = n // (nw * ch)
    rows_per_w = nchunks * ch
    row_idx = (jnp.arange(b, dtype=jnp.int32) * c)[:, None] + indices[None, :]
    idx3 = row_idx.reshape(nw, nchunks, ch)
    zf = z.reshape(n, d)
    is_id = jnp.all(indices == jnp.arange(c, dtype=indices.dtype))
    out = lax.cond(
        is_id,
        lambda: _copy_rows(n, d, nw, rows_per_w, ch=4, nb=3)(zf),
        lambda: _permute_rows(n, d, nw, nchunks, ch)(zf, idx3),
    )
    return out.reshape(b, c, d)
